# baseline (speedup 1.0000x reference)
import jax
import jax.numpy as jnp
from jax.experimental import pallas as pl
from jax.experimental.pallas import tpu as pltpu

_B = 256
_C = 4 * 128 * 128
_R = 8


def _body(t_ref, a_ref, x_ref, e_ref, o_ref):
    i = pl.program_id(0)
    row_ids = jax.lax.broadcasted_iota(jnp.int32, (_R, 1), 0)
    al = jnp.zeros((_R, 1), jnp.float32)
    sg = jnp.zeros((_R, 1), jnp.float32)
    for r in range(_R):
        a = a_ref[t_ref[i * _R + r]]
        al = jnp.where(row_ids == r, jnp.sqrt(a), al)
        sg = jnp.where(row_ids == r, jnp.sqrt(1.0 - a), sg)
    o_ref[:, :] = al * x_ref[:, :] + sg * e_ref[:, :]


def kernel(x, eps, t, alpha_sq):
    orig_shape = x.shape
    x2 = x.reshape(_B, _C)
    e2 = eps.reshape(_B, _C)
    t32 = t.astype(jnp.int32)
    grid_spec = pltpu.PrefetchScalarGridSpec(
        num_scalar_prefetch=2,
        grid=(_B // _R,),
        in_specs=[
            pl.BlockSpec((_R, _C), lambda i, t_ref, a_ref: (i, 0)),
            pl.BlockSpec((_R, _C), lambda i, t_ref, a_ref: (i, 0)),
        ],
        out_specs=pl.BlockSpec((_R, _C), lambda i, t_ref, a_ref: (i, 0)),
    )
    out = pl.pallas_call(
        _body,
        grid_spec=grid_spec,
        out_shape=jax.ShapeDtypeStruct((_B, _C), jnp.float32),
    )(t32, alpha_sq, x2, e2)
    return out.reshape(orig_shape)


# speedup vs baseline: 1.0106x; 1.0106x over previous
import jax
import jax.numpy as jnp
from jax.experimental import pallas as pl
from jax.experimental.pallas import tpu as pltpu

_B = 256
_C = 4 * 128 * 128
_R = 16


def _body(t_ref, a_ref, x_ref, e_ref, o_ref):
    i = pl.program_id(0)
    row_ids = jax.lax.broadcasted_iota(jnp.int32, (_R, 1), 0)
    al = jnp.zeros((_R, 1), jnp.float32)
    sg = jnp.zeros((_R, 1), jnp.float32)
    for r in range(_R):
        a = a_ref[t_ref[i * _R + r]]
        al = jnp.where(row_ids == r, jnp.sqrt(a), al)
        sg = jnp.where(row_ids == r, jnp.sqrt(1.0 - a), sg)
    o_ref[:, :] = al * x_ref[:, :] + sg * e_ref[:, :]


def kernel(x, eps, t, alpha_sq):
    orig_shape = x.shape
    x2 = x.reshape(_B, _C)
    e2 = eps.reshape(_B, _C)
    t32 = t.astype(jnp.int32)
    grid_spec = pltpu.PrefetchScalarGridSpec(
        num_scalar_prefetch=2,
        grid=(_B // _R,),
        in_specs=[
            pl.BlockSpec((_R, _C), lambda i, t_ref, a_ref: (i, 0)),
            pl.BlockSpec((_R, _C), lambda i, t_ref, a_ref: (i, 0)),
        ],
        out_specs=pl.BlockSpec((_R, _C), lambda i, t_ref, a_ref: (i, 0)),
    )
    out = pl.pallas_call(
        _body,
        grid_spec=grid_spec,
        out_shape=jax.ShapeDtypeStruct((_B, _C), jnp.float32),
    )(t32, alpha_sq, x2, e2)
    return out.reshape(orig_shape)


# speedup vs baseline: 4.0485x; 4.0059x over previous
import jax
import jax.numpy as jnp
from jax.experimental import pallas as pl
from jax.experimental.pallas import tpu as pltpu

_B = 256
_R = 16


def _body(t_ref, a_ref, x_ref, e_ref, o_ref):
    i = pl.program_id(0)
    row_ids = jax.lax.broadcasted_iota(jnp.int32, (_R, 1, 1, 1), 0)
    al = jnp.zeros((_R, 1, 1, 1), jnp.float32)
    sg = jnp.zeros((_R, 1, 1, 1), jnp.float32)
    for r in range(_R):
        a = a_ref[t_ref[i * _R + r]]
        al = jnp.where(row_ids == r, jnp.sqrt(a), al)
        sg = jnp.where(row_ids == r, jnp.sqrt(1.0 - a), sg)
    o_ref[...] = al * x_ref[...] + sg * e_ref[...]


def kernel(x, eps, t, alpha_sq):
    t32 = t.astype(jnp.int32)
    blk = (_R,) + x.shape[1:]
    grid_spec = pltpu.PrefetchScalarGridSpec(
        num_scalar_prefetch=2,
        grid=(_B // _R,),
        in_specs=[
            pl.BlockSpec(blk, lambda i, t_ref, a_ref: (i, 0, 0, 0)),
            pl.BlockSpec(blk, lambda i, t_ref, a_ref: (i, 0, 0, 0)),
        ],
        out_specs=pl.BlockSpec(blk, lambda i, t_ref, a_ref: (i, 0, 0, 0)),
    )
    return pl.pallas_call(
        _body,
        grid_spec=grid_spec,
        out_shape=jax.ShapeDtypeStruct(x.shape, x.dtype),
    )(t32, alpha_sq, x, eps)
